# Initial kernel scaffold; baseline (speedup 1.0000x reference)
import jax
import jax.numpy as jnp
from jax.experimental import pallas as pl


def kernel(x, edge_index, edge_attr, We1, be1, W1a, b1a, W1b, b1b, We2, be2, W2a, b2a, W2b, b2b, W3, b3, W4, b4, W5, b5, W6, b6, W7, b7, W8, b8):
    raise NotImplementedError("write your pallas kernel here")


# speedup vs baseline: 2.6273x; 2.6273x over previous
import functools

import jax
import jax.numpy as jnp
from jax import lax
from jax.experimental import pallas as pl
from jax.experimental.pallas import tpu as pltpu
from jax.experimental.pallas import tpu_sc as plsc

NC = 2
NS = 16
LANES = 16


@functools.partial(jax.jit, static_argnames=("n", "e", "w", "ch"))
def _sc_gine_aggr(x, src, dst, ea, zeros, *, n, e, w, ch):
    nw = NC * NS
    epw = e // nw
    nchunks = epw // ch
    rpt = ((n + NS - 1) // NS + 7) // 8 * 8
    last_rows = n - (NS - 1) * rpt
    assert last_rows > 0 and last_rows % 8 == 0

    mesh = plsc.VectorSubcoreMesh(core_axis_name="c", subcore_axis_name="s")

    @functools.partial(
        pl.kernel,
        out_type=jax.ShapeDtypeStruct((NC, n, w), jnp.float32),
        mesh=mesh,
        scratch_types=[
            pltpu.VMEM((ch,), jnp.int32),
            pltpu.VMEM((ch,), jnp.int32),
            pltpu.VMEM((ch, w), jnp.float32),
            pltpu.VMEM((ch, w), jnp.float32),
            pltpu.VMEM_SHARED((n, w), jnp.float32),
            pltpu.SemaphoreType.DMA,
        ],
        compiler_params=pltpu.CompilerParams(use_tc_tiling_on_sc=False),
    )
    def aggr_kernel(x_hbm, src_hbm, dst_hbm, ea_hbm, zeros_hbm, out_hbm,
                    src_v, dst_v, rows_v, ea_v, acc_sh, sem):
        c = lax.axis_index("c")
        s = lax.axis_index("s")
        wid = s * NC + c

        row0 = s * rpt

        @pl.when(s < NS - 1)
        def _zero_main():
            pltpu.sync_copy(zeros_hbm.at[pl.ds(row0, rpt)],
                            acc_sh.at[pl.ds(row0, rpt)])

        @pl.when(s == NS - 1)
        def _zero_last():
            pltpu.sync_copy(zeros_hbm.at[pl.ds((NS - 1) * rpt, last_rows)],
                            acc_sh.at[pl.ds((NS - 1) * rpt, last_rows)])

        plsc.subcore_barrier()

        base = wid * epw

        def chunk_body(i, carry):
            off = base + i * ch
            pltpu.sync_copy(src_hbm.at[pl.ds(off, ch)], src_v)
            pltpu.sync_copy(dst_hbm.at[pl.ds(off, ch)], dst_v)
            pltpu.sync_copy(ea_hbm.at[pl.ds(off, ch)], ea_v)
            pltpu.async_copy(x_hbm.at[src_v], rows_v, sem).wait()

            def edge_body(ei, cc):
                for k in range(w // LANES):
                    sl = pl.ds(k * LANES, LANES)
                    v = rows_v[ei, sl] + ea_v[ei, sl]
                    rows_v[ei, sl] = jnp.maximum(v, 0.0)
                return cc

            lax.fori_loop(0, ch, edge_body, 0)
            pltpu.sync_copy(rows_v, acc_sh.at[dst_v], add=True)
            return carry

        lax.fori_loop(0, nchunks, chunk_body, 0)
        plsc.subcore_barrier()

        @pl.when(s < NS - 1)
        def _out_main():
            pltpu.sync_copy(acc_sh.at[pl.ds(row0, rpt)],
                            out_hbm.at[c, pl.ds(row0, rpt)])

        @pl.when(s == NS - 1)
        def _out_last():
            pltpu.sync_copy(acc_sh.at[pl.ds((NS - 1) * rpt, last_rows)],
                            out_hbm.at[c, pl.ds((NS - 1) * rpt, last_rows)])

    return aggr_kernel(x, src, dst, ea, zeros)


def _tc_edge_embed(edge_attr, We1, be1, We2, be2, *, be_blk):
    e, de = edge_attr.shape
    d1 = We1.shape[1]
    d2 = We2.shape[1]

    def body(a_ref, w1_ref, b1_ref, w2_ref, b2_ref, o1_ref, o2_ref):
        a = a_ref[...]
        o1_ref[...] = jnp.dot(a, w1_ref[...],
                              preferred_element_type=jnp.float32) + b1_ref[...]
        o2_ref[...] = jnp.dot(a, w2_ref[...],
                              preferred_element_type=jnp.float32) + b2_ref[...]

    return pl.pallas_call(
        body,
        grid=(e // be_blk,),
        in_specs=[
            pl.BlockSpec((be_blk, de), lambda i: (i, 0)),
            pl.BlockSpec((de, d1), lambda i: (0, 0)),
            pl.BlockSpec((1, d1), lambda i: (0, 0)),
            pl.BlockSpec((de, d2), lambda i: (0, 0)),
            pl.BlockSpec((1, d2), lambda i: (0, 0)),
        ],
        out_specs=[
            pl.BlockSpec((be_blk, d1), lambda i: (i, 0)),
            pl.BlockSpec((be_blk, d2), lambda i: (i, 0)),
        ],
        out_shape=[
            jax.ShapeDtypeStruct((e, d1), jnp.float32),
            jax.ShapeDtypeStruct((e, d2), jnp.float32),
        ],
    )(edge_attr, We1, be1, We2, be2)


def _tc_node_mlp(x, parts, Wa, ba, Wb, bb, *, bn):
    n, d = x.shape
    dh = Wa.shape[1]
    do = Wb.shape[1]

    def body(x_ref, p_ref, wa_ref, ba_ref, wb_ref, bb_ref, o_ref):
        h = x_ref[...] + p_ref[0] + p_ref[1]
        t = jnp.maximum(
            jnp.dot(h, wa_ref[...], preferred_element_type=jnp.float32)
            + ba_ref[...], 0.0)
        o_ref[...] = jnp.dot(t, wb_ref[...],
                             preferred_element_type=jnp.float32) + bb_ref[...]

    return pl.pallas_call(
        body,
        grid=(n // bn,),
        in_specs=[
            pl.BlockSpec((bn, d), lambda i: (i, 0)),
            pl.BlockSpec((NC, bn, d), lambda i: (0, i, 0)),
            pl.BlockSpec((d, dh), lambda i: (0, 0)),
            pl.BlockSpec((1, dh), lambda i: (0, 0)),
            pl.BlockSpec((dh, do), lambda i: (0, 0)),
            pl.BlockSpec((1, do), lambda i: (0, 0)),
        ],
        out_specs=pl.BlockSpec((bn, do), lambda i: (i, 0)),
        out_shape=jax.ShapeDtypeStruct((n, do), jnp.float32),
    )(x, parts, Wa, ba, Wb, bb)


def _tc_mlp_pool_heads(h1, parts, W2a, b2a, W2b, b2b, W3, b3, W4, b4,
                       W5, b5, W6, b6, W7, b7, W8, b8, *, bn):
    n, d = h1.shape
    sc = W6.shape[1]
    tc = W8.shape[1]

    def body(h_ref, p_ref, w2a_ref, b2a_ref, w2b_ref, b2b_ref,
             w3_ref, b3_ref, w4_ref, b4_ref, w5_ref, b5_ref, w6_ref, b6_ref,
             w7_ref, b7_ref, w8_ref, b8_ref, s_ref, t_ref, acc_ref):
        i = pl.program_id(0)

        @pl.when(i == 0)
        def _init():
            acc_ref[...] = jnp.zeros_like(acc_ref)

        h2 = h_ref[...] + p_ref[0] + p_ref[1]
        z = jnp.maximum(
            jnp.dot(h2, w2a_ref[...], preferred_element_type=jnp.float32)
            + b2a_ref[...], 0.0)
        z = jnp.dot(z, w2b_ref[...],
                    preferred_element_type=jnp.float32) + b2b_ref[...]
        acc_ref[...] += jnp.sum(z, axis=0, keepdims=True)

        @pl.when(i == pl.num_programs(0) - 1)
        def _heads():
            g = acc_ref[...] * (1.0 / n)
            z3 = jnp.maximum(
                jnp.dot(g, w3_ref[...], preferred_element_type=jnp.float32)
                + b3_ref[...], 0.0)
            z4 = jnp.maximum(
                jnp.dot(z3, w4_ref[...], preferred_element_type=jnp.float32)
                + b4_ref[...], 0.0)
            zs = jnp.maximum(
                jnp.dot(z4, w5_ref[...], preferred_element_type=jnp.float32)
                + b5_ref[...], 0.0)
            s_ref[...] = jax.nn.sigmoid(
                jnp.dot(zs, w6_ref[...], preferred_element_type=jnp.float32)
                + b6_ref[...])
            zt = jnp.maximum(
                jnp.dot(z4, w7_ref[...], preferred_element_type=jnp.float32)
                + b7_ref[...], 0.0)
            t_ref[...] = jax.nn.sigmoid(
                jnp.dot(zt, w8_ref[...], preferred_element_type=jnp.float32)
                + b8_ref[...])

    full = lambda shape: pl.BlockSpec(shape, lambda i: tuple(0 for _ in shape))
    return pl.pallas_call(
        body,
        grid=(n // bn,),
        in_specs=[
            pl.BlockSpec((bn, d), lambda i: (i, 0)),
            pl.BlockSpec((NC, bn, d), lambda i: (0, i, 0)),
            full(W2a.shape), full((1, b2a.shape[-1])),
            full(W2b.shape), full((1, b2b.shape[-1])),
            full(W3.shape), full((1, b3.shape[-1])),
            full(W4.shape), full((1, b4.shape[-1])),
            full(W5.shape), full((1, b5.shape[-1])),
            full(W6.shape), full((1, b6.shape[-1])),
            full(W7.shape), full((1, b7.shape[-1])),
            full(W8.shape), full((1, b8.shape[-1])),
        ],
        out_specs=[
            pl.BlockSpec((1, sc), lambda i: (0, 0)),
            pl.BlockSpec((1, tc), lambda i: (0, 0)),
        ],
        out_shape=[
            jax.ShapeDtypeStruct((1, sc), jnp.float32),
            jax.ShapeDtypeStruct((1, tc), jnp.float32),
        ],
        scratch_shapes=[pltpu.VMEM((1, d), jnp.float32)],
    )(h1, parts, W2a, b2a.reshape(1, -1), W2b, b2b.reshape(1, -1),
      W3, b3.reshape(1, -1), W4, b4.reshape(1, -1), W5, b5.reshape(1, -1),
      W6, b6.reshape(1, -1), W7, b7.reshape(1, -1), W8, b8.reshape(1, -1))


def kernel(x, edge_index, edge_attr, We1, be1, W1a, b1a, W1b, b1b,
           We2, be2, W2a, b2a, W2b, b2b, W3, b3, W4, b4, W5, b5, W6, b6,
           W7, b7, W8, b8):
    n, d = x.shape
    e = edge_attr.shape[0]
    src = edge_index[0]
    dst = edge_index[1]

    ea1, ea2 = _tc_edge_embed(edge_attr, We1, be1.reshape(1, -1),
                              We2, be2.reshape(1, -1), be_blk=4000)

    zeros128 = jnp.zeros((n, d), jnp.float32)
    p1 = _sc_gine_aggr(x, src, dst, ea1, zeros128, n=n, e=e, w=d, ch=80)
    h1 = _tc_node_mlp(x, p1, W1a, b1a.reshape(1, -1), W1b, b1b.reshape(1, -1),
                      bn=2000)

    d2 = h1.shape[1]
    zeros64 = jnp.zeros((n, d2), jnp.float32)
    p2 = _sc_gine_aggr(h1, src, dst, ea2, zeros64, n=n, e=e, w=d2, ch=80)

    s, t = _tc_mlp_pool_heads(h1, p2, W2a, b2a, W2b, b2b, W3, b3, W4, b4,
                              W5, b5, W6, b6, W7, b7, W8, b8, bn=2000)
    return (s.reshape(-1), t.reshape(-1))


# speedup vs baseline: 4.4343x; 1.6878x over previous
import functools

import jax
import jax.numpy as jnp
from jax import lax
from jax.experimental import pallas as pl
from jax.experimental.pallas import tpu as pltpu
from jax.experimental.pallas import tpu_sc as plsc

NC = 2
NS = 16
LANES = 16


@functools.partial(jax.jit, static_argnames=("n", "e", "w", "ch"))
def _sc_gine_aggr(x, src, dst, ea, zeros, *, n, e, w, ch):
    nw = NC * NS
    epw = e // nw
    nchunks = epw // ch
    npairs = nchunks // 2
    tail = nchunks % 2
    rpt = ((n + NS - 1) // NS + 7) // 8 * 8
    last_rows = n - (NS - 1) * rpt
    assert last_rows > 0 and last_rows % 8 == 0

    mesh = plsc.VectorSubcoreMesh(core_axis_name="c", subcore_axis_name="s")

    @functools.partial(
        pl.kernel,
        out_type=jax.ShapeDtypeStruct((NC, n, w), jnp.float32),
        mesh=mesh,
        scratch_types=[
            pltpu.VMEM((2, ch), jnp.int32),
            pltpu.VMEM((2, ch), jnp.int32),
            pltpu.VMEM((2, ch), jnp.int32),
            pltpu.VMEM((2, ch, w), jnp.float32),
            pltpu.VMEM((2, ch, w), jnp.float32),
            pltpu.VMEM_SHARED((n, w), jnp.float32),
            pltpu.SemaphoreType.DMA,
            pltpu.SemaphoreType.DMA,
            pltpu.SemaphoreType.DMA,
            pltpu.SemaphoreType.DMA,
            pltpu.SemaphoreType.DMA,
            pltpu.SemaphoreType.DMA,
        ],
        compiler_params=pltpu.CompilerParams(use_tc_tiling_on_sc=False),
    )
    def aggr_kernel(x_hbm, src_hbm, dst_hbm, ea_hbm, zeros_hbm, out_hbm,
                    src_v, dst_v, dsc_v, rows_v, ea_v, acc_sh,
                    sem_l0, sem_l1, sem_g0, sem_g1, sem_s0, sem_s1):
        c = lax.axis_index("c")
        s = lax.axis_index("s")
        wid = s * NC + c
        sems_l = (sem_l0, sem_l1)
        sems_g = (sem_g0, sem_g1)
        sems_s = (sem_s0, sem_s1)

        row0 = s * rpt

        @pl.when(s < NS - 1)
        def _zero_main():
            pltpu.sync_copy(zeros_hbm.at[pl.ds(row0, rpt)],
                            acc_sh.at[pl.ds(row0, rpt)])

        @pl.when(s == NS - 1)
        def _zero_last():
            pltpu.sync_copy(zeros_hbm.at[pl.ds((NS - 1) * rpt, last_rows)],
                            acc_sh.at[pl.ds((NS - 1) * rpt, last_rows)])

        plsc.subcore_barrier()

        base = wid * epw

        def start_lin(ci, slot):
            off = base + ci * ch
            pltpu.async_copy(src_hbm.at[pl.ds(off, ch)], src_v.at[slot],
                             sems_l[slot])
            pltpu.async_copy(dst_hbm.at[pl.ds(off, ch)], dst_v.at[slot],
                             sems_l[slot])
            pltpu.async_copy(ea_hbm.at[pl.ds(off, ch)], ea_v.at[slot],
                             sems_l[slot])

        def wait_lin(slot):
            pltpu.make_async_copy(src_hbm.at[pl.ds(0, ch)], src_v.at[slot],
                                  sems_l[slot]).wait()
            pltpu.make_async_copy(dst_hbm.at[pl.ds(0, ch)], dst_v.at[slot],
                                  sems_l[slot]).wait()
            pltpu.make_async_copy(ea_hbm.at[pl.ds(0, ch)], ea_v.at[slot],
                                  sems_l[slot]).wait()

        def start_gather(slot):
            pltpu.async_copy(x_hbm.at[src_v.at[slot]], rows_v.at[slot],
                             sems_g[slot])

        def wait_gather(slot):
            pltpu.make_async_copy(x_hbm.at[pl.ds(0, ch)], rows_v.at[slot],
                                  sems_g[slot]).wait()

        def start_scatter(slot):
            for j in range(ch // LANES):
                sl = pl.ds(j * LANES, LANES)
                dsc_v[slot, sl] = dst_v[slot, sl]
            pltpu.async_copy(rows_v.at[slot], acc_sh.at[dsc_v.at[slot]],
                             sems_s[slot], add=True)

        def wait_scatter(slot):
            pltpu.make_async_copy(x_hbm.at[pl.ds(0, ch)], rows_v.at[slot],
                                  sems_s[slot]).wait()

        def compute(slot):
            def edge_body(ei, cc):
                for k in range(w // LANES):
                    sl = pl.ds(k * LANES, LANES)
                    v = rows_v[slot, ei, sl] + ea_v[slot, ei, sl]
                    rows_v[slot, ei, sl] = jnp.maximum(v, 0.0)
                return cc

            lax.fori_loop(0, ch, edge_body, 0)

        start_lin(0, 0)
        start_lin(1, 1)
        wait_lin(0)
        start_gather(0)

        def pair_body(i2, carry):
            a = 2 * i2
            wait_gather(0)
            wait_lin(1)

            @pl.when(i2 > 0)
            def _drain_s1():
                wait_scatter(1)

            start_gather(1)
            compute(0)
            start_scatter(0)

            @pl.when(a + 2 < nchunks)
            def _lin_next0():
                start_lin(a + 2, 0)

            wait_gather(1)
            wait_scatter(0)

            @pl.when(a + 2 < nchunks)
            def _gather_next0():
                wait_lin(0)
                start_gather(0)

            compute(1)
            start_scatter(1)

            @pl.when(a + 3 < nchunks)
            def _lin_next1():
                start_lin(a + 3, 1)

            return carry

        lax.fori_loop(0, npairs, pair_body, 0)

        if tail:
            wait_gather(0)
            compute(0)
            start_scatter(0)
        wait_scatter(0)
        wait_scatter(1)
        plsc.subcore_barrier()

        @pl.when(s < NS - 1)
        def _out_main():
            pltpu.sync_copy(acc_sh.at[pl.ds(row0, rpt)],
                            out_hbm.at[c, pl.ds(row0, rpt)])

        @pl.when(s == NS - 1)
        def _out_last():
            pltpu.sync_copy(acc_sh.at[pl.ds((NS - 1) * rpt, last_rows)],
                            out_hbm.at[c, pl.ds((NS - 1) * rpt, last_rows)])

    return aggr_kernel(x, src, dst, ea, zeros)


def _tc_edge_embed(edge_attr, We1, be1, We2, be2, *, be_blk):
    e, de = edge_attr.shape
    d1 = We1.shape[1]
    d2 = We2.shape[1]

    def body(a_ref, w1_ref, b1_ref, w2_ref, b2_ref, o1_ref, o2_ref):
        a = a_ref[...]
        o1_ref[...] = jnp.dot(a, w1_ref[...],
                              preferred_element_type=jnp.float32) + b1_ref[...]
        o2_ref[...] = jnp.dot(a, w2_ref[...],
                              preferred_element_type=jnp.float32) + b2_ref[...]

    return pl.pallas_call(
        body,
        grid=(e // be_blk,),
        in_specs=[
            pl.BlockSpec((be_blk, de), lambda i: (i, 0)),
            pl.BlockSpec((de, d1), lambda i: (0, 0)),
            pl.BlockSpec((1, d1), lambda i: (0, 0)),
            pl.BlockSpec((de, d2), lambda i: (0, 0)),
            pl.BlockSpec((1, d2), lambda i: (0, 0)),
        ],
        out_specs=[
            pl.BlockSpec((be_blk, d1), lambda i: (i, 0)),
            pl.BlockSpec((be_blk, d2), lambda i: (i, 0)),
        ],
        out_shape=[
            jax.ShapeDtypeStruct((e, d1), jnp.float32),
            jax.ShapeDtypeStruct((e, d2), jnp.float32),
        ],
    )(edge_attr, We1, be1, We2, be2)


def _tc_node_mlp(x, parts, Wa, ba, Wb, bb, *, bn):
    n, d = x.shape
    dh = Wa.shape[1]
    do = Wb.shape[1]

    def body(x_ref, p_ref, wa_ref, ba_ref, wb_ref, bb_ref, o_ref):
        h = x_ref[...] + p_ref[0] + p_ref[1]
        t = jnp.maximum(
            jnp.dot(h, wa_ref[...], preferred_element_type=jnp.float32)
            + ba_ref[...], 0.0)
        o_ref[...] = jnp.dot(t, wb_ref[...],
                             preferred_element_type=jnp.float32) + bb_ref[...]

    return pl.pallas_call(
        body,
        grid=(n // bn,),
        in_specs=[
            pl.BlockSpec((bn, d), lambda i: (i, 0)),
            pl.BlockSpec((NC, bn, d), lambda i: (0, i, 0)),
            pl.BlockSpec((d, dh), lambda i: (0, 0)),
            pl.BlockSpec((1, dh), lambda i: (0, 0)),
            pl.BlockSpec((dh, do), lambda i: (0, 0)),
            pl.BlockSpec((1, do), lambda i: (0, 0)),
        ],
        out_specs=pl.BlockSpec((bn, do), lambda i: (i, 0)),
        out_shape=jax.ShapeDtypeStruct((n, do), jnp.float32),
    )(x, parts, Wa, ba, Wb, bb)


def _tc_mlp_pool_heads(h1, parts, W2a, b2a, W2b, b2b, W3, b3, W4, b4,
                       W5, b5, W6, b6, W7, b7, W8, b8, *, bn):
    n, d = h1.shape
    sc = W6.shape[1]
    tc = W8.shape[1]

    def body(h_ref, p_ref, w2a_ref, b2a_ref, w2b_ref, b2b_ref,
             w3_ref, b3_ref, w4_ref, b4_ref, w5_ref, b5_ref, w6_ref, b6_ref,
             w7_ref, b7_ref, w8_ref, b8_ref, s_ref, t_ref, acc_ref):
        i = pl.program_id(0)

        @pl.when(i == 0)
        def _init():
            acc_ref[...] = jnp.zeros_like(acc_ref)

        h2 = h_ref[...] + p_ref[0] + p_ref[1]
        z = jnp.maximum(
            jnp.dot(h2, w2a_ref[...], preferred_element_type=jnp.float32)
            + b2a_ref[...], 0.0)
        z = jnp.dot(z, w2b_ref[...],
                    preferred_element_type=jnp.float32) + b2b_ref[...]
        acc_ref[...] += jnp.sum(z, axis=0, keepdims=True)

        @pl.when(i == pl.num_programs(0) - 1)
        def _heads():
            g = acc_ref[...] * (1.0 / n)
            z3 = jnp.maximum(
                jnp.dot(g, w3_ref[...], preferred_element_type=jnp.float32)
                + b3_ref[...], 0.0)
            z4 = jnp.maximum(
                jnp.dot(z3, w4_ref[...], preferred_element_type=jnp.float32)
                + b4_ref[...], 0.0)
            zs = jnp.maximum(
                jnp.dot(z4, w5_ref[...], preferred_element_type=jnp.float32)
                + b5_ref[...], 0.0)
            s_ref[...] = jax.nn.sigmoid(
                jnp.dot(zs, w6_ref[...], preferred_element_type=jnp.float32)
                + b6_ref[...])
            zt = jnp.maximum(
                jnp.dot(z4, w7_ref[...], preferred_element_type=jnp.float32)
                + b7_ref[...], 0.0)
            t_ref[...] = jax.nn.sigmoid(
                jnp.dot(zt, w8_ref[...], preferred_element_type=jnp.float32)
                + b8_ref[...])

    full = lambda shape: pl.BlockSpec(shape, lambda i: tuple(0 for _ in shape))
    return pl.pallas_call(
        body,
        grid=(n // bn,),
        in_specs=[
            pl.BlockSpec((bn, d), lambda i: (i, 0)),
            pl.BlockSpec((NC, bn, d), lambda i: (0, i, 0)),
            full(W2a.shape), full((1, b2a.shape[-1])),
            full(W2b.shape), full((1, b2b.shape[-1])),
            full(W3.shape), full((1, b3.shape[-1])),
            full(W4.shape), full((1, b4.shape[-1])),
            full(W5.shape), full((1, b5.shape[-1])),
            full(W6.shape), full((1, b6.shape[-1])),
            full(W7.shape), full((1, b7.shape[-1])),
            full(W8.shape), full((1, b8.shape[-1])),
        ],
        out_specs=[
            pl.BlockSpec((1, sc), lambda i: (0, 0)),
            pl.BlockSpec((1, tc), lambda i: (0, 0)),
        ],
        out_shape=[
            jax.ShapeDtypeStruct((1, sc), jnp.float32),
            jax.ShapeDtypeStruct((1, tc), jnp.float32),
        ],
        scratch_shapes=[pltpu.VMEM((1, d), jnp.float32)],
    )(h1, parts, W2a, b2a.reshape(1, -1), W2b, b2b.reshape(1, -1),
      W3, b3.reshape(1, -1), W4, b4.reshape(1, -1), W5, b5.reshape(1, -1),
      W6, b6.reshape(1, -1), W7, b7.reshape(1, -1), W8, b8.reshape(1, -1))


def kernel(x, edge_index, edge_attr, We1, be1, W1a, b1a, W1b, b1b,
           We2, be2, W2a, b2a, W2b, b2b, W3, b3, W4, b4, W5, b5, W6, b6,
           W7, b7, W8, b8):
    n, d = x.shape
    e = edge_attr.shape[0]
    src = edge_index[0]
    dst = edge_index[1]

    ea1, ea2 = _tc_edge_embed(edge_attr, We1, be1.reshape(1, -1),
                              We2, be2.reshape(1, -1), be_blk=4000)

    zeros128 = jnp.zeros((n, d), jnp.float32)
    p1 = _sc_gine_aggr(x, src, dst, ea1, zeros128, n=n, e=e, w=d, ch=80)
    h1 = _tc_node_mlp(x, p1, W1a, b1a.reshape(1, -1), W1b, b1b.reshape(1, -1),
                      bn=2000)

    d2 = h1.shape[1]
    zeros64 = jnp.zeros((n, d2), jnp.float32)
    p2 = _sc_gine_aggr(h1, src, dst, ea2, zeros64, n=n, e=e, w=d2, ch=80)

    s, t = _tc_mlp_pool_heads(h1, p2, W2a, b2a, W2b, b2b, W3, b3, W4, b4,
                              W5, b5, W6, b6, W7, b7, W8, b8, bn=2000)
    return (s.reshape(-1), t.reshape(-1))
